# baseline (speedup 1.0000x reference)
import functools

import jax
import jax.numpy as jnp
from jax import lax
from jax.experimental import pallas as pl
from jax.experimental.pallas import tpu as pltpu
from jax.experimental.pallas import tpu_sc as plsc

EMBED_DIM = 64
_LANES = 16
_LANE_TILE = 128

_mesh = plsc.VectorSubcoreMesh(
    core_axis_name="c", subcore_axis_name="s", num_cores=1
)


@functools.partial(
    pl.kernel,
    mesh=_mesh,
    out_type=jax.ShapeDtypeStruct((EMBED_DIM,), jnp.float32),
    scratch_types=[
        pltpu.VMEM((_LANES,), jnp.int32),
        pltpu.VMEM((EMBED_DIM, _LANE_TILE), jnp.float32),
        pltpu.VMEM((EMBED_DIM,), jnp.float32),
    ],
    compiler_params=pltpu.CompilerParams(needs_layout_passes=False),
)
def _sc_lookup(ids_hbm, tableT_hbm, out_hbm, idx_v, blk_v, col_v):
    s = lax.axis_index("s")

    @pl.when(s == 0)
    def _():
        pltpu.sync_copy(ids_hbm.at[0, pl.ds(0, _LANES)], idx_v)
        idx0 = idx_v[...][0]
        cbase = pl.multiple_of((idx0 // _LANE_TILE) * _LANE_TILE, _LANE_TILE)
        pltpu.sync_copy(tableT_hbm.at[:, pl.ds(cbase, _LANE_TILE)], blk_v)
        c = jnp.full((_LANES,), idx0 % _LANE_TILE, jnp.int32)
        for i in range(EMBED_DIM // _LANES):
            rows = lax.iota(jnp.int32, _LANES) + i * _LANES
            col_v[pl.ds(i * _LANES, _LANES)] = plsc.load_gather(blk_v, [rows, c])
        pltpu.sync_copy(col_v, out_hbm)


def kernel(ids, table):
    ids16 = lax.slice(ids, (0, 0), (1, 16)).astype(jnp.int32)
    return _sc_lookup(ids16, table.T)
